# Initial kernel scaffold; baseline (speedup 1.0000x reference)
import jax
import jax.numpy as jnp
from jax.experimental import pallas as pl


def kernel(p, fc_pos_W, fc_pos_b, blocks_fc0_W, blocks_fc0_b, blocks_fc1_W, blocks_fc1_b, blocks_sc_W, fc_c_W, fc_c_b):
    raise NotImplementedError("write your pallas kernel here")


# speedup vs baseline: 2.5886x; 2.5886x over previous
import functools

import jax
import jax.numpy as jnp
from jax.experimental import pallas as pl
from jax.experimental.pallas import tpu as pltpu

RESO = 32
PAD = 0.1
HID = 128


def _block_kernel(x_ref, w0_ref, b0_ref, w1_ref, b1_ref, sc_ref, o_ref):
    x = x_ref[...]
    net = jnp.maximum(x, 0.0) @ w0_ref[...] + b0_ref[...]
    dx = jnp.maximum(net, 0.0) @ w1_ref[...] + b1_ref[...]
    o_ref[...] = x @ sc_ref[...] + dx


def _block(x, w0, b0, w1, b1, sc):
    N, K = x.shape
    TN = 2048
    b0 = b0.reshape(1, HID)
    b1 = b1.reshape(1, HID)
    return pl.pallas_call(
        _block_kernel,
        grid=(N // TN,),
        in_specs=[
            pl.BlockSpec((TN, K), lambda i: (i, 0)),
            pl.BlockSpec((K, HID), lambda i: (0, 0)),
            pl.BlockSpec((1, HID), lambda i: (0, 0)),
            pl.BlockSpec((HID, HID), lambda i: (0, 0)),
            pl.BlockSpec((1, HID), lambda i: (0, 0)),
            pl.BlockSpec((K, HID), lambda i: (0, 0)),
        ],
        out_specs=pl.BlockSpec((TN, HID), lambda i: (i, 0)),
        out_shape=jax.ShapeDtypeStruct((N, HID), jnp.float32),
    )(x, w0, b0, w1, b1, sc)


def _matmul_kernel(x_ref, w_ref, b_ref, o_ref):
    o_ref[...] = x_ref[...] @ w_ref[...] + b_ref[...]


def _matmul_bias(x, w, b):
    N, K = x.shape
    M = w.shape[1]
    TN = 2048
    b = b.reshape(1, M)
    return pl.pallas_call(
        _matmul_kernel,
        grid=(N // TN,),
        in_specs=[
            pl.BlockSpec((TN, K), lambda i: (i, 0)),
            pl.BlockSpec((K, M), lambda i: (0, 0)),
            pl.BlockSpec((1, M), lambda i: (0, 0)),
        ],
        out_specs=pl.BlockSpec((TN, M), lambda i: (i, 0)),
        out_shape=jax.ShapeDtypeStruct((N, M), jnp.float32),
    )(x, w, b)


def kernel(p, fc_pos_W, fc_pos_b, blocks_fc0_W, blocks_fc0_b, blocks_fc1_W,
           blocks_fc1_b, blocks_sc_W, fc_c_W, fc_c_b):
    B, T, _ = p.shape
    NB = blocks_fc0_W.shape[0]
    nseg = B * RESO ** 3

    pn = jnp.clip(p / (1.0 + PAD + 1e-3) + 0.5, 0.0, 1.0 - 1e-3)
    gi = (pn * RESO).astype(jnp.int32)
    idx = gi[..., 0] + RESO * (gi[..., 1] + RESO * gi[..., 2])
    flat_idx = (idx + jnp.arange(B, dtype=idx.dtype)[:, None] * (RESO ** 3)).reshape(-1)

    pf = p.reshape(B * T, 3)
    net = _matmul_bias(pf, fc_pos_W, fc_pos_b)
    net = _block(net, blocks_fc0_W[0], blocks_fc0_b[0], blocks_fc1_W[0],
                 blocks_fc1_b[0], blocks_sc_W[0])

    for i in range(1, NB):
        seg = jax.ops.segment_max(net, flat_idx, num_segments=nseg)
        seg = jnp.where(jnp.isfinite(seg), seg, 0.0)
        pooled = seg[flat_idx]
        net = jnp.concatenate([net, pooled], axis=-1)
        net = _block(net, blocks_fc0_W[i], blocks_fc0_b[i], blocks_fc1_W[i],
                     blocks_fc1_b[i], blocks_sc_W[i])

    c = _matmul_bias(net, fc_c_W, fc_c_b)
    CDIM = c.shape[-1]

    ones = jnp.ones((B * T,), jnp.float32)
    cnt = jax.ops.segment_sum(ones, flat_idx, num_segments=nseg)
    sums = jax.ops.segment_sum(c, flat_idx, num_segments=nseg)
    mean = sums / jnp.maximum(cnt, 1.0)[:, None]
    fea = mean.reshape(B, RESO ** 3, CDIM).transpose(0, 2, 1).reshape(
        B, CDIM, RESO, RESO, RESO)

    mask = (cnt.reshape(B, RESO, RESO, RESO) > 0.0)
    return fea, mask
